# Initial kernel scaffold; baseline (speedup 1.0000x reference)
import jax
import jax.numpy as jnp
from jax.experimental import pallas as pl


def kernel(x, edge_index, edge_index_global, edge_weight_global, W_in, b_in, W_layers, b_layers, W_out, b_out, alpha, gamma):
    raise NotImplementedError("write your pallas kernel here")


# speedup vs baseline: 4.1573x; 4.1573x over previous
import functools

import jax
import jax.numpy as jnp
from jax import lax
from jax.experimental import pallas as pl
from jax.experimental.pallas import tpu as pltpu
from jax.experimental.pallas import tpu_sc as plsc

N = 10000
E = 320000
D = 128
C = 64
L = 24
NCLS = 10

NC = 2
NS = 16
NW = NC * NS
LN = 16

NP = 10240
RPT = NP // NS
CH = 128
ECHUNKS = E // CH
NODECH = RPT // CH

_mesh = plsc.VectorSubcoreMesh(
    core_axis_name="c", subcore_axis_name="s", num_cores=NC, num_subcores=NS)


@functools.partial(
    pl.kernel,
    out_type=jax.ShapeDtypeStruct((NC, NP), jnp.float32),
    mesh=_mesh,
    scratch_types=[
        pltpu.VMEM_SHARED((NP,), jnp.float32),
        pltpu.VMEM((CH,), jnp.int32),
        pltpu.VMEM((CH,), jnp.float32),
        pltpu.VMEM((RPT,), jnp.float32),
    ],
)
def _sc_degree(dst_hbm, out_hbm, acc, didx, ones_v, slice_v):
    cid = lax.axis_index("c")
    sid = lax.axis_index("s")
    wid = sid * NC + cid
    base = sid * RPT

    def _z(i, carry):
        slice_v[pl.ds(i * LN, LN)] = jnp.zeros((LN,), jnp.float32)
        return carry
    lax.fori_loop(0, RPT // LN, _z, 0)
    pltpu.sync_copy(slice_v, acc.at[pl.ds(base, RPT)])

    def _o(i, carry):
        ones_v[pl.ds(i * LN, LN)] = jnp.ones((LN,), jnp.float32)
        return carry
    lax.fori_loop(0, CH // LN, _o, 0)
    plsc.subcore_barrier()

    nch = (ECHUNKS - wid + NW - 1) // NW

    def _body(i, carry):
        off = (wid + i * NW) * CH
        pltpu.sync_copy(dst_hbm.at[pl.ds(off, CH)], didx)
        pltpu.sync_copy(ones_v, acc.at[didx], add=True)
        return carry
    lax.fori_loop(0, nch, _body, 0)
    plsc.subcore_barrier()

    pltpu.sync_copy(acc.at[pl.ds(base, RPT)], slice_v)
    pltpu.sync_copy(slice_v, out_hbm.at[cid, pl.ds(base, RPT)])


@functools.partial(
    pl.kernel,
    out_type=jax.ShapeDtypeStruct((NC, NP, C), jnp.float32),
    mesh=_mesh,
    scratch_types=[
        pltpu.VMEM_SHARED((NP, C), jnp.float32),
        pltpu.VMEM_SHARED((NP, C), jnp.float32),
        pltpu.VMEM((CH,), jnp.int32),
        pltpu.VMEM((CH,), jnp.int32),
        pltpu.VMEM((CH,), jnp.float32),
        pltpu.VMEM((CH, C), jnp.float32),
        pltpu.VMEM((CH, C), jnp.float32),
        pltpu.VMEM((CH, C), jnp.float32),
        pltpu.VMEM((RPT,), jnp.float32),
        pltpu.SemaphoreType.DMA,
    ],
    compiler_params=pltpu.CompilerParams(use_tc_tiling_on_sc=False),
)
def _sc_aggregate(h_hbm, srcl, dstl, srcg, dstg, wg_hbm, invd_hbm,
                  out_hbm, accl, accg, sidx, didx, wbuf, rows, lbuf, gbuf,
                  invd, sem):
    cid = lax.axis_index("c")
    sid = lax.axis_index("s")
    wid = sid * NC + cid
    base = sid * RPT

    def _z(r, carry):
        for j in range(C // LN):
            rows[r, pl.ds(j * LN, LN)] = jnp.zeros((LN,), jnp.float32)
        return carry
    lax.fori_loop(0, CH, _z, 0)
    for k in range(NODECH):
        pltpu.sync_copy(rows, accl.at[pl.ds(base + k * CH, CH)])
        pltpu.sync_copy(rows, accg.at[pl.ds(base + k * CH, CH)])
    plsc.subcore_barrier()

    nch = (ECHUNKS - wid + NW - 1) // NW

    def _lbody(i, carry):
        off = (wid + i * NW) * CH
        pltpu.sync_copy(srcl.at[pl.ds(off, CH)], sidx)
        pltpu.sync_copy(dstl.at[pl.ds(off, CH)], didx)
        pltpu.async_copy(h_hbm.at[sidx], rows, sem).wait()
        pltpu.sync_copy(rows, accl.at[didx], add=True)
        return carry
    lax.fori_loop(0, nch, _lbody, 0)

    def _gbody(i, carry):
        off = (wid + i * NW) * CH
        pltpu.sync_copy(srcg.at[pl.ds(off, CH)], sidx)
        pltpu.sync_copy(dstg.at[pl.ds(off, CH)], didx)
        pltpu.sync_copy(wg_hbm.at[pl.ds(off, CH)], wbuf)
        pltpu.async_copy(h_hbm.at[sidx], rows, sem).wait()

        def _scale(k, carry2):
            w16 = wbuf[pl.ds(k * LN, LN)]
            for i in range(LN):
                r = k * LN + i
                w = w16[i]
                for j in range(C // LN):
                    rows[r, pl.ds(j * LN, LN)] = rows[r, pl.ds(j * LN, LN)] * w
            return carry2
        lax.fori_loop(0, CH // LN, _scale, 0)
        pltpu.sync_copy(rows, accg.at[didx], add=True)
        return carry
    lax.fori_loop(0, nch, _gbody, 0)
    plsc.subcore_barrier()

    pltpu.sync_copy(invd_hbm.at[pl.ds(base, RPT)], invd)
    for k in range(NODECH):
        rb = base + k * CH
        pltpu.sync_copy(accl.at[pl.ds(rb, CH)], lbuf)
        pltpu.sync_copy(accg.at[pl.ds(rb, CH)], gbuf)

        def _comb(k2, carry):
            s16 = invd[pl.ds(k * CH + k2 * LN, LN)]
            for i in range(LN):
                r = k2 * LN + i
                s = s16[i]
                for j in range(C // LN):
                    rows[r, pl.ds(j * LN, LN)] = (
                        lbuf[r, pl.ds(j * LN, LN)] * s
                        + gbuf[r, pl.ds(j * LN, LN)])
            return carry
        lax.fori_loop(0, CH // LN, _comb, 0)
        pltpu.sync_copy(rows, out_hbm.at[cid, pl.ds(rb, CH)])


def _pre_body(x_ref, w_ref, b_ref, deg_ref, ewg_ref, sc_ref,
              h0_ref, ax0_ref, invd_ref, wg_ref):
    alpha = sc_ref[0, 0]
    gamma = sc_ref[0, 1]
    a1 = 1.0 - alpha
    h0 = jnp.dot(x_ref[...], w_ref[...],
                 preferred_element_type=jnp.float32) + b_ref[...]
    h0_ref[...] = h0
    ax0_ref[...] = alpha * h0
    d = deg_ref[0] + deg_ref[1]
    invd_ref[...] = a1 / jnp.maximum(d, 1.0)
    wg_ref[...] = (a1 * gamma) * ewg_ref[...]


_tc_pre = pl.pallas_call(
    _pre_body,
    out_shape=(
        jax.ShapeDtypeStruct((NP, C), jnp.float32),
        jax.ShapeDtypeStruct((NP, C), jnp.float32),
        jax.ShapeDtypeStruct((NP // 128, 128), jnp.float32),
        jax.ShapeDtypeStruct((ECHUNKS, 128), jnp.float32),
    ),
    in_specs=[
        pl.BlockSpec(memory_space=pltpu.VMEM),
        pl.BlockSpec(memory_space=pltpu.VMEM),
        pl.BlockSpec(memory_space=pltpu.VMEM),
        pl.BlockSpec(memory_space=pltpu.VMEM),
        pl.BlockSpec(memory_space=pltpu.VMEM),
        pl.BlockSpec(memory_space=pltpu.SMEM),
    ],
)


def _layer_body(p_ref, ax0_ref, w_ref, b_ref, h_ref):
    hp = p_ref[0] + p_ref[1] + ax0_ref[...]
    h = jnp.dot(hp, w_ref[...], preferred_element_type=jnp.float32) + b_ref[...]
    h_ref[...] = jnp.maximum(h, 0.0)


_tc_layer = pl.pallas_call(
    _layer_body,
    out_shape=jax.ShapeDtypeStruct((NP, C), jnp.float32),
)


def _out_body(h_ref, w_ref, b_ref, o_ref):
    logits = jnp.dot(h_ref[:N], w_ref[...],
                     preferred_element_type=jnp.float32) + b_ref[...]
    m = jnp.max(logits, axis=1, keepdims=True)
    z = logits - m
    o_ref[...] = z - jnp.log(jnp.sum(jnp.exp(z), axis=1, keepdims=True))


_tc_out = pl.pallas_call(
    _out_body,
    out_shape=jax.ShapeDtypeStruct((N, NCLS), jnp.float32),
)


def kernel(x, edge_index, edge_index_global, edge_weight_global,
           W_in, b_in, W_layers, b_layers, W_out, b_out, alpha, gamma):
    srcl = edge_index[0]
    dstl = edge_index[1]
    srcg = edge_index_global[0]
    dstg = edge_index_global[1]

    degp = _sc_degree(dstl)

    xp = jnp.pad(x, ((0, NP - N), (0, 0)))
    scal = jnp.stack([alpha, gamma]).reshape(1, 2)
    deg2d = degp.reshape(NC, NP // 128, 128)
    ew2d = edge_weight_global.reshape(ECHUNKS, 128)
    h0, ax0, invd2d, wg2d = _tc_pre(xp, W_in, b_in.reshape(1, C), deg2d,
                                    ew2d, scal)
    invd = invd2d.reshape(NP)
    wg = wg2d.reshape(E)

    h = h0
    for i in range(L - 2):
        part = _sc_aggregate(h, srcl, dstl, srcg, dstg, wg, invd)
        h = _tc_layer(part, ax0, W_layers[i], b_layers[i].reshape(1, C))

    return _tc_out(h, W_out, b_out.reshape(1, NCLS))
